# Initial kernel scaffold; baseline (speedup 1.0000x reference)
import jax
import jax.numpy as jnp
from jax.experimental import pallas as pl


def kernel(x, edge_index, h, c, params):
    raise NotImplementedError("write your pallas kernel here")


# speedup vs baseline: 4.2910x; 4.2910x over previous
import functools

import jax
import jax.numpy as jnp
from jax import lax
from jax.experimental import pallas as pl
from jax.experimental.pallas import tpu as pltpu
from jax.experimental.pallas import tpu_sc as plsc

_N = 50000
_E = 800000
_C_IN = 8
_HID = 32
_T_IN = 6
_NLAYERS = 2

_NP = 50176
_NTILES = 16
_EB = 128
_NB = 391
_EPT = _NB * _EB
_EPAD = _NTILES * _EPT
_RPT = _NP // _NTILES
_ZCH = 224
_BN = 256


def _sc_prop_body(gidx, sidx, table, out, gidx_v, sidx_v, rows_v, zero_v, acc):
    cid = lax.axis_index("c")
    tid = lax.axis_index("s")
    row0 = tid * _RPT

    def zfill(i, _):
        zero_v[i] = jnp.zeros((16,), jnp.float32)
        return 0
    lax.fori_loop(0, _ZCH, zfill, 0)

    def zchunk(k, _):
        pltpu.sync_copy(zero_v, acc.at[pl.ds(row0 + k * _ZCH, _ZCH)])
        return 0
    lax.fori_loop(0, _RPT // _ZCH, zchunk, 0)
    plsc.subcore_barrier()

    tbl = table.at[cid]

    def ebody(j, _):
        pltpu.sync_copy(gidx.at[tid, j], gidx_v)
        pltpu.sync_copy(sidx.at[tid, j], sidx_v)
        pltpu.sync_copy(tbl.at[gidx_v], rows_v)
        pltpu.sync_copy(rows_v, acc.at[sidx_v], add=True)
        return 0
    lax.fori_loop(0, _NB, ebody, 0)
    plsc.subcore_barrier()

    pltpu.sync_copy(acc.at[pl.ds(row0, _RPT)], out.at[cid, pl.ds(row0, _RPT)])


@jax.jit
def _sc_prop(gidx, sidx, table):
    mesh = plsc.VectorSubcoreMesh(core_axis_name="c", subcore_axis_name="s")
    f = pl.kernel(
        _sc_prop_body,
        mesh=mesh,
        compiler_params=pltpu.CompilerParams(use_tc_tiling_on_sc=False),
        out_type=jax.ShapeDtypeStruct((2, _NP, 16), jnp.float32),
        scratch_types=[
            pltpu.VMEM((_EB,), jnp.int32),
            pltpu.VMEM((_EB,), jnp.int32),
            pltpu.VMEM((_EB, 16), jnp.float32),
            pltpu.VMEM((_ZCH, 16), jnp.float32),
            pltpu.VMEM_SHARED((_NP, 16), jnp.float32),
        ],
    )
    return f(gidx, sidx, table)


def _prep_body(deg_ref, h_ref, x_ref, dinv16_ref, mdinv2_ref, uh_ref,
               ux1_ref, ux2_ref):
    deg = deg_ref[:, 0:1]
    dinv = jnp.where(deg > 0.0, lax.rsqrt(jnp.where(deg > 0.0, deg, 1.0)), 0.0)
    d16 = jnp.broadcast_to(dinv, (deg.shape[0], 16))
    dinv16_ref[...] = d16
    mdinv2_ref[...] = -(d16 * d16)
    hb = h_ref[...]
    uh_ref[0] = d16 * hb[:, :16]
    uh_ref[1] = d16 * hb[:, 16:]
    xb = x_ref[...]
    ux1_ref[0] = d16 * xb[:, 0:16]
    ux1_ref[1] = d16 * xb[:, 16:32]
    ux2_ref[0] = d16 * xb[:, 32:48]
    ux2_ref[1] = jnp.zeros_like(d16)


@jax.jit
def _tc_prep(deg16, h_pad, x48_pad):
    nb = _NP // _BN
    return pl.pallas_call(
        _prep_body,
        grid=(nb,),
        in_specs=[
            pl.BlockSpec((_BN, 16), lambda i: (i, 0)),
            pl.BlockSpec((_BN, _HID), lambda i: (i, 0)),
            pl.BlockSpec((_BN, 48), lambda i: (i, 0)),
        ],
        out_specs=[
            pl.BlockSpec((_BN, 16), lambda i: (i, 0)),
            pl.BlockSpec((_BN, 16), lambda i: (i, 0)),
            pl.BlockSpec((2, _BN, 16), lambda i: (0, i, 0)),
            pl.BlockSpec((2, _BN, 16), lambda i: (0, i, 0)),
            pl.BlockSpec((2, _BN, 16), lambda i: (0, i, 0)),
        ],
        out_shape=[
            jax.ShapeDtypeStruct((_NP, 16), jnp.float32),
            jax.ShapeDtypeStruct((_NP, 16), jnp.float32),
            jax.ShapeDtypeStruct((2, _NP, 16), jnp.float32),
            jax.ShapeDtypeStruct((2, _NP, 16), jnp.float32),
            jax.ShapeDtypeStruct((2, _NP, 16), jnp.float32),
        ],
    )(deg16, h_pad, x48_pad)


def _scale_body(g_ref, m_ref, u_ref):
    m = m_ref[...]
    u_ref[0] = m * g_ref[0]
    u_ref[1] = m * g_ref[1]


@jax.jit
def _tc_scale(g1, mdinv2):
    nb = _NP // _BN
    return pl.pallas_call(
        _scale_body,
        grid=(nb,),
        in_specs=[
            pl.BlockSpec((2, _BN, 16), lambda i: (0, i, 0)),
            pl.BlockSpec((_BN, 16), lambda i: (i, 0)),
        ],
        out_specs=pl.BlockSpec((2, _BN, 16), lambda i: (0, i, 0)),
        out_shape=jax.ShapeDtypeStruct((2, _NP, 16), jnp.float32),
    )(g1, mdinv2)


def _cell_body(x_ref, o1x_ref, o2x_ref, h_ref, g1h_ref, g2h_ref, c_ref,
               d16_ref, wx_ref, wh_ref, b_ref, wc_ref,
               hout_ref, cout_ref, uout_ref):
    d16 = d16_ref[...]
    d8 = d16[:, :8]
    xin = jnp.concatenate(
        [x_ref[0], d8 * o1x_ref[0], d8 * o2x_ref[0]], axis=1)
    hb = h_ref[...]
    hin = jnp.concatenate(
        [hb,
         d16 * g1h_ref[0], d16 * g1h_ref[1],
         d16 * g2h_ref[0], d16 * g2h_ref[1]], axis=1)
    z = (jnp.dot(xin, wx_ref[...], preferred_element_type=jnp.float32)
         + jnp.dot(hin, wh_ref[...], preferred_element_type=jnp.float32)
         + b_ref[...])
    cb = c_ref[...]
    wc = wc_ref[...]
    gi = jax.nn.sigmoid(z[:, 0:32] + wc[0:1, :] * cb)
    gf = jax.nn.sigmoid(z[:, 32:64] + wc[1:2, :] * cb)
    gt = jnp.tanh(z[:, 64:96])
    cn = gf * cb + gi * gt
    go = jax.nn.sigmoid(z[:, 96:128] + wc[2:3, :] * cn)
    hn = go * jnp.tanh(cn)
    hout_ref[...] = hn
    cout_ref[...] = cn
    uout_ref[0] = d16 * hn[:, :16]
    uout_ref[1] = d16 * hn[:, 16:]


@functools.partial(jax.jit, static_argnums=(0,))
def _tc_cell(t, x48, o1x, o2x, h, g1h, g2h, c, dinv16, wx, wh, b, wc):
    nb = _NP // _BN
    return pl.pallas_call(
        _cell_body,
        grid=(nb,),
        in_specs=[
            pl.BlockSpec((1, _BN, 8), lambda i: (t, i, 0)),
            pl.BlockSpec((1, _BN, 8), lambda i: (t, i, 0)),
            pl.BlockSpec((1, _BN, 8), lambda i: (t, i, 0)),
            pl.BlockSpec((_BN, _HID), lambda i: (i, 0)),
            pl.BlockSpec((2, _BN, 16), lambda i: (0, i, 0)),
            pl.BlockSpec((2, _BN, 16), lambda i: (0, i, 0)),
            pl.BlockSpec((_BN, _HID), lambda i: (i, 0)),
            pl.BlockSpec((_BN, 16), lambda i: (i, 0)),
            pl.BlockSpec((24, 128), lambda i: (0, 0)),
            pl.BlockSpec((96, 128), lambda i: (0, 0)),
            pl.BlockSpec((1, 128), lambda i: (0, 0)),
            pl.BlockSpec((3, _HID), lambda i: (0, 0)),
        ],
        out_specs=[
            pl.BlockSpec((_BN, _HID), lambda i: (i, 0)),
            pl.BlockSpec((_BN, _HID), lambda i: (i, 0)),
            pl.BlockSpec((2, _BN, 16), lambda i: (0, i, 0)),
        ],
        out_shape=[
            jax.ShapeDtypeStruct((_NP, _HID), jnp.float32),
            jax.ShapeDtypeStruct((_NP, _HID), jnp.float32),
            jax.ShapeDtypeStruct((2, _NP, 16), jnp.float32),
        ],
    )(x48, o1x, o2x, h, g1h, g2h, c, dinv16, wx, wh, b, wc)


def _cat_weights(p, prefix, cin):
    cols = []
    for g in ("i", "f", "c", "o"):
        w = p["W" + prefix + "_" + g]
        cols.append(jnp.concatenate([w[0] - w[2], -w[1], -2.0 * w[2]], axis=0))
    return jnp.concatenate(cols, axis=1)


def kernel(x, edge_index, h, c, params):
    p = params
    src = edge_index[0].astype(jnp.int32)
    dst = edge_index[1].astype(jnp.int32)
    self_m = src == dst
    n_i32 = jnp.int32(_N)
    srcg = jnp.where(self_m, n_i32, src)
    dstg = jnp.where(self_m, n_i32, dst)

    padi = jnp.full((_EPAD - _E,), n_i32, jnp.int32)
    def lay(a):
        return jnp.concatenate([a, padi]).reshape(_NTILES, _NB, _EB)
    gidx_p, sidx_p = lay(srcg), lay(dst)
    gidx_d, sidx_d = lay(dstg), lay(src)

    h_pad = jnp.pad(h, ((0, _NP - _N), (0, 0)))
    c_pad = jnp.pad(c, ((0, _NP - _N), (0, 0)))
    x48 = x.reshape(_N, _C_IN, _T_IN).transpose(0, 2, 1).reshape(_N, 48)
    x48_pad = jnp.pad(x48, ((0, _NP - _N), (0, 0)))

    ones_tab = jnp.pad(jnp.ones((2, _N, 16), jnp.float32),
                       ((0, 0), (0, _NP - _N), (0, 0)))

    deg_out = _sc_prop(gidx_d, sidx_d, ones_tab)
    dinv16, mdinv2, uh, ux1, ux2 = _tc_prep(deg_out[0], h_pad, x48_pad)

    g1xa = _sc_prop(gidx_p, sidx_p, ux1)
    g2xa = _sc_prop(gidx_p, sidx_p, _tc_scale(g1xa, mdinv2))
    g1xb = _sc_prop(gidx_p, sidx_p, ux2)
    g2xb = _sc_prop(gidx_p, sidx_p, _tc_scale(g1xb, mdinv2))

    def to6(ga, gb):
        flat = jnp.concatenate([ga[0], ga[1], gb[0]], axis=1)
        return flat.reshape(_NP, _T_IN, _C_IN).transpose(1, 0, 2)
    o1x = to6(g1xa, g1xb)
    o2x = to6(g2xa, g2xb)
    x6 = x48_pad.reshape(_NP, _T_IN, _C_IN).transpose(1, 0, 2)

    wx = _cat_weights(p, "x", _C_IN)
    wh = _cat_weights(p, "h", _HID)
    b = jnp.concatenate([p["bx_" + g] + p["bh_" + g] + p["b_" + g]
                         for g in ("i", "f", "c", "o")]).reshape(1, 128)
    wc = jnp.stack([p["w_c_i"], p["w_c_f"], p["w_c_o"]])

    h_cur, c_cur, u_cur = h_pad, c_pad, uh
    hs, cs = [], []
    for t in range(_T_IN):
        for _ in range(_NLAYERS):
            g1h = _sc_prop(gidx_p, sidx_p, u_cur)
            g2h = _sc_prop(gidx_p, sidx_p, _tc_scale(g1h, mdinv2))
            h_cur, c_cur, u_cur = _tc_cell(
                t, x6, o1x, o2x, h_cur, g1h, g2h, c_cur,
                dinv16, wx, wh, b, wc)
        hs.append(h_cur[:_N])
        cs.append(c_cur[:_N])
    return jnp.stack(hs), jnp.stack(cs)


# speedup vs baseline: 12.7671x; 2.9753x over previous
import functools

import jax
import jax.numpy as jnp
from jax import lax
from jax.experimental import pallas as pl
from jax.experimental.pallas import tpu as pltpu
from jax.experimental.pallas import tpu_sc as plsc

_N = 50000
_E = 800000
_C_IN = 8
_HID = 32
_T_IN = 6
_NLAYERS = 2

_NP = 50176
_NTILES = 16
_EB = 128
_NB = 396
_EPT = _NB * _EB
_EPAD = _NTILES * _EPT
_W = 12
_RPT = _NP // _NTILES
_BN = 512


def _sc_prop_body(idx, zeros, table, out, iq, rows, isem, gsem, ssem, acc):
    cid = lax.axis_index("c")
    tid = lax.axis_index("s")
    row0 = tid * _RPT
    tbl = table.at[cid]

    pltpu.sync_copy(zeros.at[pl.ds(row0, _RPT)], acc.at[pl.ds(row0, _RPT)])
    plsc.subcore_barrier()

    _edge_pass(idx, tid, tbl, acc, iq, rows, isem, gsem, ssem)
    plsc.subcore_barrier()

    pltpu.sync_copy(acc.at[pl.ds(row0, _RPT)], out.at[cid, pl.ds(row0, _RPT)])


def _edge_pass(idx, tid, src, tgt, iq, rows, isem, gsem, ssem):
    @pl.loop(0, _NB // _W)
    def _egroup(g):
        j0 = g * _W
        pltpu.async_copy(idx.at[tid, pl.ds(j0, _W)], iq, isem.at[0]).wait()
        gds = [pltpu.async_copy(src.at[iq.at[k, 0]],
                                rows.at[pl.ds(k * _EB, _EB)], gsem.at[k])
               for k in range(_W)]
        sds = []
        for k in range(_W):
            gds[k].wait()
            sds.append(pltpu.async_copy(rows.at[pl.ds(k * _EB, _EB)],
                                        tgt.at[iq.at[k, 1]], ssem.at[k],
                                        add=True))
        for d in sds:
            d.wait()


def _sc_prop2_body(idx, zeros, table, scale, g1, u2, g2,
                   iq, rows, sbuf, isem, gsem, ssem, acc):
    cid = lax.axis_index("c")
    tid = lax.axis_index("s")
    row0 = tid * _RPT
    tbl = table.at[cid]

    pltpu.sync_copy(zeros.at[pl.ds(row0, _RPT)], acc.at[pl.ds(row0, _RPT)])
    plsc.subcore_barrier()
    _edge_pass(idx, tid, tbl, acc, iq, rows, isem, gsem, ssem)
    plsc.subcore_barrier()

    for c0, cn in [(k * 512, 512) for k in range(_RPT // 512)] + \
                  [(_RPT - _RPT % 512, _RPT % 512)]:
        if cn == 0:
            continue
        r0 = row0 + c0
        pltpu.sync_copy(acc.at[pl.ds(r0, cn)], rows.at[pl.ds(0, cn)])
        pltpu.sync_copy(rows.at[pl.ds(0, cn)], g1.at[cid, pl.ds(r0, cn)])
        pltpu.sync_copy(scale.at[pl.ds(r0, cn)], sbuf.at[pl.ds(0, cn)])

        def smul(i, _):
            rows[i] = rows[i] * sbuf[i]
            return 0
        lax.fori_loop(0, cn, smul, 0)
        pltpu.sync_copy(rows.at[pl.ds(0, cn)], u2.at[cid, pl.ds(r0, cn)])
        pltpu.sync_copy(zeros.at[pl.ds(r0, cn)], acc.at[pl.ds(r0, cn)])
    plsc.subcore_barrier()

    _edge_pass(idx, tid, u2.at[cid], acc, iq, rows, isem, gsem, ssem)
    plsc.subcore_barrier()
    pltpu.sync_copy(acc.at[pl.ds(row0, _RPT)], g2.at[cid, pl.ds(row0, _RPT)])


@jax.jit
def _sc_prop2(idx, zeros, table, scale):
    mesh = plsc.VectorSubcoreMesh(core_axis_name="c", subcore_axis_name="s")
    f = pl.kernel(
        _sc_prop2_body,
        mesh=mesh,
        compiler_params=pltpu.CompilerParams(use_tc_tiling_on_sc=False),
        out_type=[jax.ShapeDtypeStruct((2, _NP, 16), jnp.float32),
                  jax.ShapeDtypeStruct((2, _NP, 16), jnp.float32),
                  jax.ShapeDtypeStruct((2, _NP, 16), jnp.float32)],
        scratch_types=[
            pltpu.VMEM((_W, 2, _EB), jnp.int32),
            pltpu.VMEM((_W * _EB, 16), jnp.float32),
            pltpu.VMEM((512, 16), jnp.float32),
            pltpu.SemaphoreType.DMA((1,)),
            pltpu.SemaphoreType.DMA((_W,)),
            pltpu.SemaphoreType.DMA((_W,)),
            pltpu.VMEM_SHARED((_NP, 16), jnp.float32),
        ],
    )
    g1, _, g2 = f(idx, zeros, table, scale)
    return g1, g2


@jax.jit
def _sc_prop(idx, zeros, table):
    mesh = plsc.VectorSubcoreMesh(core_axis_name="c", subcore_axis_name="s")
    f = pl.kernel(
        _sc_prop_body,
        mesh=mesh,
        compiler_params=pltpu.CompilerParams(use_tc_tiling_on_sc=False),
        out_type=jax.ShapeDtypeStruct((2, _NP, 16), jnp.float32),
        scratch_types=[
            pltpu.VMEM((_W, 2, _EB), jnp.int32),
            pltpu.VMEM((_W * _EB, 16), jnp.float32),
            pltpu.SemaphoreType.DMA((1,)),
            pltpu.SemaphoreType.DMA((_W,)),
            pltpu.SemaphoreType.DMA((_W,)),
            pltpu.VMEM_SHARED((_NP, 16), jnp.float32),
        ],
    )
    return f(idx, zeros, table)


def _prep_body(deg_ref, h_ref, x_ref, dinv16_ref, mdinv2_ref, uh_ref,
               ux1_ref, ux2_ref):
    deg = deg_ref[:, 0:1]
    dinv = jnp.where(deg > 0.0, lax.rsqrt(jnp.where(deg > 0.0, deg, 1.0)), 0.0)
    d16 = jnp.broadcast_to(dinv, (deg.shape[0], 16))
    dinv16_ref[...] = d16
    mdinv2_ref[...] = -(d16 * d16)
    hb = h_ref[...]
    uh_ref[0] = d16 * hb[:, :16]
    uh_ref[1] = d16 * hb[:, 16:]
    xb = x_ref[...]
    ux1_ref[0] = d16 * xb[:, 0:16]
    ux1_ref[1] = d16 * xb[:, 16:32]
    ux2_ref[0] = d16 * xb[:, 32:48]
    ux2_ref[1] = jnp.zeros_like(d16)


@jax.jit
def _tc_prep(deg16, h_pad, x48_pad):
    nb = _NP // _BN
    return pl.pallas_call(
        _prep_body,
        grid=(nb,),
        in_specs=[
            pl.BlockSpec((_BN, 16), lambda i: (i, 0)),
            pl.BlockSpec((_BN, _HID), lambda i: (i, 0)),
            pl.BlockSpec((_BN, 48), lambda i: (i, 0)),
        ],
        out_specs=[
            pl.BlockSpec((_BN, 16), lambda i: (i, 0)),
            pl.BlockSpec((_BN, 16), lambda i: (i, 0)),
            pl.BlockSpec((2, _BN, 16), lambda i: (0, i, 0)),
            pl.BlockSpec((2, _BN, 16), lambda i: (0, i, 0)),
            pl.BlockSpec((2, _BN, 16), lambda i: (0, i, 0)),
        ],
        out_shape=[
            jax.ShapeDtypeStruct((_NP, 16), jnp.float32),
            jax.ShapeDtypeStruct((_NP, 16), jnp.float32),
            jax.ShapeDtypeStruct((2, _NP, 16), jnp.float32),
            jax.ShapeDtypeStruct((2, _NP, 16), jnp.float32),
            jax.ShapeDtypeStruct((2, _NP, 16), jnp.float32),
        ],
    )(deg16, h_pad, x48_pad)


def _cell_body(x_ref, o1x_ref, o2x_ref, h_ref, g1h_ref, g2h_ref, c_ref,
               d16_ref, wx_ref, wh_ref, b_ref, wc_ref,
               hout_ref, cout_ref, uout_ref):
    d16 = d16_ref[...]
    d8 = d16[:, :8]

    def dot(a, w):
        return jnp.dot(a, w, preferred_element_type=jnp.float32)

    z = (dot(x_ref[0], wx_ref[0:8])
         + dot(d8 * o1x_ref[0], wx_ref[8:16])
         + dot(d8 * o2x_ref[0], wx_ref[16:24])
         + dot(h_ref[...], wh_ref[0:32])
         + dot(d16 * g1h_ref[0], wh_ref[32:48])
         + dot(d16 * g1h_ref[1], wh_ref[48:64])
         + dot(d16 * g2h_ref[0], wh_ref[64:80])
         + dot(d16 * g2h_ref[1], wh_ref[80:96])
         + b_ref[...])
    cb = c_ref[...]
    wc = wc_ref[...]
    gi = jax.nn.sigmoid(z[:, 0:32] + wc[0:1, :] * cb)
    gf = jax.nn.sigmoid(z[:, 32:64] + wc[1:2, :] * cb)
    gt = jnp.tanh(z[:, 64:96])
    cn = gf * cb + gi * gt
    go = jax.nn.sigmoid(z[:, 96:128] + wc[2:3, :] * cn)
    hn = go * jnp.tanh(cn)
    hout_ref[...] = hn
    cout_ref[...] = cn
    uout_ref[0] = d16 * hn[:, :16]
    uout_ref[1] = d16 * hn[:, 16:]


@functools.partial(jax.jit, static_argnums=(0,))
def _tc_cell(t, x48, o1x, o2x, h, g1h, g2h, c, dinv16, wx, wh, b, wc):
    nb = _NP // _BN
    return pl.pallas_call(
        _cell_body,
        grid=(nb,),
        in_specs=[
            pl.BlockSpec((1, _BN, 8), lambda i: (t, i, 0)),
            pl.BlockSpec((1, _BN, 8), lambda i: (t, i, 0)),
            pl.BlockSpec((1, _BN, 8), lambda i: (t, i, 0)),
            pl.BlockSpec((_BN, _HID), lambda i: (i, 0)),
            pl.BlockSpec((2, _BN, 16), lambda i: (0, i, 0)),
            pl.BlockSpec((2, _BN, 16), lambda i: (0, i, 0)),
            pl.BlockSpec((_BN, _HID), lambda i: (i, 0)),
            pl.BlockSpec((_BN, 16), lambda i: (i, 0)),
            pl.BlockSpec((24, 128), lambda i: (0, 0)),
            pl.BlockSpec((96, 128), lambda i: (0, 0)),
            pl.BlockSpec((1, 128), lambda i: (0, 0)),
            pl.BlockSpec((3, _HID), lambda i: (0, 0)),
        ],
        out_specs=[
            pl.BlockSpec((_BN, _HID), lambda i: (i, 0)),
            pl.BlockSpec((_BN, _HID), lambda i: (i, 0)),
            pl.BlockSpec((2, _BN, 16), lambda i: (0, i, 0)),
        ],
        out_shape=[
            jax.ShapeDtypeStruct((_NP, _HID), jnp.float32),
            jax.ShapeDtypeStruct((_NP, _HID), jnp.float32),
            jax.ShapeDtypeStruct((2, _NP, 16), jnp.float32),
        ],
    )(x48, o1x, o2x, h, g1h, g2h, c, dinv16, wx, wh, b, wc)


def _cat_weights(p, prefix, cin):
    cols = []
    for g in ("i", "f", "c", "o"):
        w = p["W" + prefix + "_" + g]
        cols.append(jnp.concatenate([w[0] - w[2], -w[1], -2.0 * w[2]], axis=0))
    return jnp.concatenate(cols, axis=1)


def kernel(x, edge_index, h, c, params):
    p = params
    src = edge_index[0].astype(jnp.int32)
    dst = edge_index[1].astype(jnp.int32)
    self_m = src == dst
    n_i32 = jnp.int32(_N)
    srcg = jnp.where(self_m, n_i32, src)
    dstg = jnp.where(self_m, n_i32, dst)

    npad_e = _EPAD - _E
    padg = jnp.full((npad_e,), n_i32, jnp.int32)
    pads = (n_i32 + 1 + (jnp.arange(npad_e, dtype=jnp.int32)
                         % jnp.int32(_NP - _N - 1)))
    def lay(g, s):
        g = jnp.concatenate([g, padg]).reshape(_NTILES, _NB, 1, _EB)
        s = jnp.concatenate([s, pads]).reshape(_NTILES, _NB, 1, _EB)
        return jnp.concatenate([g, s], axis=2)
    idx_p = lay(srcg, dst)
    idx_d = lay(dstg, src)
    zz = jnp.zeros((_NP, 16), jnp.float32)

    h_pad = jnp.pad(h, ((0, _NP - _N), (0, 0)))
    c_pad = jnp.pad(c, ((0, _NP - _N), (0, 0)))
    x48 = x.reshape(_N, _C_IN, _T_IN).transpose(0, 2, 1).reshape(_N, 48)
    x48_pad = jnp.pad(x48, ((0, _NP - _N), (0, 0)))

    ones_tab = jnp.pad(jnp.ones((2, _N, 16), jnp.float32),
                       ((0, 0), (0, _NP - _N), (0, 0)))

    deg_out = _sc_prop(idx_d, zz, ones_tab)
    dinv16, mdinv2, uh, ux1, ux2 = _tc_prep(deg_out[0], h_pad, x48_pad)

    g1xa, g2xa = _sc_prop2(idx_p, zz, ux1, mdinv2)
    g1xb, g2xb = _sc_prop2(idx_p, zz, ux2, mdinv2)

    def to6(ga, gb):
        flat = jnp.concatenate([ga[0], ga[1], gb[0]], axis=1)
        return flat.reshape(_NP, _T_IN, _C_IN).transpose(1, 0, 2)
    o1x = to6(g1xa, g1xb)
    o2x = to6(g2xa, g2xb)
    x6 = x48_pad.reshape(_NP, _T_IN, _C_IN).transpose(1, 0, 2)

    wx = _cat_weights(p, "x", _C_IN)
    wh = _cat_weights(p, "h", _HID)
    b = jnp.concatenate([p["bx_" + g] + p["bh_" + g] + p["b_" + g]
                         for g in ("i", "f", "c", "o")]).reshape(1, 128)
    wc = jnp.stack([p["w_c_i"], p["w_c_f"], p["w_c_o"]])

    h_cur, c_cur, u_cur = h_pad, c_pad, uh
    hs, cs = [], []
    for t in range(_T_IN):
        for _ in range(_NLAYERS):
            g1h, g2h = _sc_prop2(idx_p, zz, u_cur, mdinv2)
            h_cur, c_cur, u_cur = _tc_cell(
                t, x6, o1x, o2x, h_cur, g1h, g2h, c_cur,
                dinv16, wx, wh, b, wc)
        hs.append(h_cur[:_N])
        cs.append(c_cur[:_N])
    return jnp.stack(hs), jnp.stack(cs)


# speedup vs baseline: 13.6642x; 1.0703x over previous
import functools

import jax
import jax.numpy as jnp
from jax import lax
from jax.experimental import pallas as pl
from jax.experimental.pallas import tpu as pltpu
from jax.experimental.pallas import tpu_sc as plsc

_N = 50000
_E = 800000
_C_IN = 8
_HID = 32
_T_IN = 6
_NLAYERS = 2

_NP = 50176
_NTILES = 16
_EB = 128
_NB = 396
_EPT = _NB * _EB
_EPAD = _NTILES * _EPT
_W = 12
_RPT = _NP // _NTILES
_BN = 1024


def _sc_prop_body(idx, zeros, table, out, iq, rows, isem, gsem, ssem, acc):
    cid = lax.axis_index("c")
    tid = lax.axis_index("s")
    row0 = tid * _RPT
    tbl = table.at[cid]

    pltpu.sync_copy(zeros.at[pl.ds(row0, _RPT)], acc.at[pl.ds(row0, _RPT)])
    plsc.subcore_barrier()

    _edge_pass(idx, tid, tbl, acc, iq, rows, isem, gsem, ssem)
    plsc.subcore_barrier()

    pltpu.sync_copy(acc.at[pl.ds(row0, _RPT)], out.at[cid, pl.ds(row0, _RPT)])


def _edge_pass(idx, tid, src, tgt, iq, rows, isem, gsem, ssem):
    @pl.loop(0, _NB // _W)
    def _egroup(g):
        j0 = g * _W
        pltpu.async_copy(idx.at[tid, pl.ds(j0, _W)], iq, isem.at[0]).wait()
        gds = [pltpu.async_copy(src.at[iq.at[k, 0]],
                                rows.at[pl.ds(k * _EB, _EB)], gsem.at[k])
               for k in range(_W)]
        sds = []
        for k in range(_W):
            gds[k].wait()
            sds.append(pltpu.async_copy(rows.at[pl.ds(k * _EB, _EB)],
                                        tgt.at[iq.at[k, 1]], ssem.at[k],
                                        add=True))
        for d in sds:
            d.wait()


def _sc_prop2_body(idx, zeros, table, scale, g1, u2, g2,
                   iq, rows, sbuf, isem, gsem, ssem, acc):
    cid = lax.axis_index("c")
    tid = lax.axis_index("s")
    row0 = tid * _RPT
    tbl = table.at[cid]

    pltpu.sync_copy(zeros.at[pl.ds(row0, _RPT)], acc.at[pl.ds(row0, _RPT)])
    plsc.subcore_barrier()
    _edge_pass(idx, tid, tbl, acc, iq, rows, isem, gsem, ssem)
    plsc.subcore_barrier()

    ch = 512
    chunks = [(k * ch, ch) for k in range(_RPT // ch)]
    if _RPT % ch:
        chunks.append((_RPT - _RPT % ch, _RPT % ch))
    outs = {0: [], 1: []}
    for i, (c0, cn) in enumerate(chunks):
        p = i & 1
        for d in outs[p]:
            d.wait()
        r0 = row0 + c0
        d1 = pltpu.async_copy(acc.at[pl.ds(r0, cn)],
                              rows.at[pl.ds(p * ch, cn)], gsem.at[p])
        dg = pltpu.async_copy(acc.at[pl.ds(r0, cn)],
                              g1.at[cid, pl.ds(r0, cn)], gsem.at[6 + p])
        d2 = pltpu.async_copy(scale.at[pl.ds(r0, cn)],
                              sbuf.at[pl.ds(p * ch, cn)], gsem.at[2 + p])
        d1.wait()
        dg.wait()
        dz = pltpu.async_copy(zeros.at[pl.ds(r0, cn)],
                              acc.at[pl.ds(r0, cn)], gsem.at[4 + p])
        d2.wait()

        def smul(i2, _):
            rows[p * ch + i2] = rows[p * ch + i2] * sbuf[p * ch + i2]
            return 0
        lax.fori_loop(0, cn, smul, 0)
        outs[p] = [
            pltpu.async_copy(rows.at[pl.ds(p * ch, cn)],
                             u2.at[cid, pl.ds(r0, cn)], ssem.at[2 + p]),
            dz,
        ]
    for p in (0, 1):
        for d in outs[p]:
            d.wait()
    plsc.subcore_barrier()

    _edge_pass(idx, tid, u2.at[cid], acc, iq, rows, isem, gsem, ssem)
    plsc.subcore_barrier()
    pltpu.sync_copy(acc.at[pl.ds(row0, _RPT)], g2.at[cid, pl.ds(row0, _RPT)])


@jax.jit
def _sc_prop2(idx, zeros, table, scale):
    mesh = plsc.VectorSubcoreMesh(core_axis_name="c", subcore_axis_name="s")
    f = pl.kernel(
        _sc_prop2_body,
        mesh=mesh,
        compiler_params=pltpu.CompilerParams(use_tc_tiling_on_sc=False),
        out_type=[jax.ShapeDtypeStruct((2, _NP, 16), jnp.float32),
                  jax.ShapeDtypeStruct((2, _NP, 16), jnp.float32),
                  jax.ShapeDtypeStruct((2, _NP, 16), jnp.float32)],
        scratch_types=[
            pltpu.VMEM((_W, 2, _EB), jnp.int32),
            pltpu.VMEM((_W * _EB, 16), jnp.float32),
            pltpu.VMEM((1024, 16), jnp.float32),
            pltpu.SemaphoreType.DMA((1,)),
            pltpu.SemaphoreType.DMA((_W,)),
            pltpu.SemaphoreType.DMA((_W,)),
            pltpu.VMEM_SHARED((_NP, 16), jnp.float32),
        ],
    )
    g1, _, g2 = f(idx, zeros, table, scale)
    return g1, g2


@jax.jit
def _sc_prop(idx, zeros, table):
    mesh = plsc.VectorSubcoreMesh(core_axis_name="c", subcore_axis_name="s")
    f = pl.kernel(
        _sc_prop_body,
        mesh=mesh,
        compiler_params=pltpu.CompilerParams(use_tc_tiling_on_sc=False),
        out_type=jax.ShapeDtypeStruct((2, _NP, 16), jnp.float32),
        scratch_types=[
            pltpu.VMEM((_W, 2, _EB), jnp.int32),
            pltpu.VMEM((_W * _EB, 16), jnp.float32),
            pltpu.SemaphoreType.DMA((1,)),
            pltpu.SemaphoreType.DMA((_W,)),
            pltpu.SemaphoreType.DMA((_W,)),
            pltpu.VMEM_SHARED((_NP, 16), jnp.float32),
        ],
    )
    return f(idx, zeros, table)


def _prep_body(deg_ref, h_ref, x_ref, dinv16_ref, mdinv2_ref, uh_ref,
               ux1_ref, ux2_ref):
    deg = deg_ref[:, 0:1]
    dinv = jnp.where(deg > 0.0, lax.rsqrt(jnp.where(deg > 0.0, deg, 1.0)), 0.0)
    d16 = jnp.broadcast_to(dinv, (deg.shape[0], 16))
    dinv16_ref[...] = d16
    mdinv2_ref[...] = -(d16 * d16)
    hb = h_ref[...]
    uh_ref[0] = d16 * hb[:, :16]
    uh_ref[1] = d16 * hb[:, 16:]
    xb = x_ref[...]
    ux1_ref[0] = d16 * xb[:, 0:16]
    ux1_ref[1] = d16 * xb[:, 16:32]
    ux2_ref[0] = d16 * xb[:, 32:48]
    ux2_ref[1] = jnp.zeros_like(d16)


@jax.jit
def _tc_prep(deg16, h_pad, x48_pad):
    nb = _NP // _BN
    return pl.pallas_call(
        _prep_body,
        grid=(nb,),
        in_specs=[
            pl.BlockSpec((_BN, 16), lambda i: (i, 0)),
            pl.BlockSpec((_BN, _HID), lambda i: (i, 0)),
            pl.BlockSpec((_BN, 48), lambda i: (i, 0)),
        ],
        out_specs=[
            pl.BlockSpec((_BN, 16), lambda i: (i, 0)),
            pl.BlockSpec((_BN, 16), lambda i: (i, 0)),
            pl.BlockSpec((2, _BN, 16), lambda i: (0, i, 0)),
            pl.BlockSpec((2, _BN, 16), lambda i: (0, i, 0)),
            pl.BlockSpec((2, _BN, 16), lambda i: (0, i, 0)),
        ],
        out_shape=[
            jax.ShapeDtypeStruct((_NP, 16), jnp.float32),
            jax.ShapeDtypeStruct((_NP, 16), jnp.float32),
            jax.ShapeDtypeStruct((2, _NP, 16), jnp.float32),
            jax.ShapeDtypeStruct((2, _NP, 16), jnp.float32),
            jax.ShapeDtypeStruct((2, _NP, 16), jnp.float32),
        ],
    )(deg16, h_pad, x48_pad)


def _cell_body(x_ref, o1x_ref, o2x_ref, h_ref, g1h_ref, g2h_ref, c_ref,
               d16_ref, wx_ref, wh_ref, b_ref, wc_ref,
               hout_ref, cout_ref, uout_ref):
    d16 = d16_ref[...]
    d8 = d16[:, :8]

    def dot(a, w):
        return jnp.dot(a, w, preferred_element_type=jnp.float32)

    z = (dot(x_ref[0], wx_ref[0:8])
         + dot(d8 * o1x_ref[0], wx_ref[8:16])
         + dot(d8 * o2x_ref[0], wx_ref[16:24])
         + dot(h_ref[...], wh_ref[0:32])
         + dot(d16 * g1h_ref[0], wh_ref[32:48])
         + dot(d16 * g1h_ref[1], wh_ref[48:64])
         + dot(d16 * g2h_ref[0], wh_ref[64:80])
         + dot(d16 * g2h_ref[1], wh_ref[80:96])
         + b_ref[...])
    cb = c_ref[...]
    wc = wc_ref[...]
    gi = jax.nn.sigmoid(z[:, 0:32] + wc[0:1, :] * cb)
    gf = jax.nn.sigmoid(z[:, 32:64] + wc[1:2, :] * cb)
    gt = jnp.tanh(z[:, 64:96])
    cn = gf * cb + gi * gt
    go = jax.nn.sigmoid(z[:, 96:128] + wc[2:3, :] * cn)
    hn = go * jnp.tanh(cn)
    hout_ref[...] = hn
    cout_ref[...] = cn
    uout_ref[0] = d16 * hn[:, :16]
    uout_ref[1] = d16 * hn[:, 16:]


@functools.partial(jax.jit, static_argnums=(0,))
def _tc_cell(t, x48, o1x, o2x, h, g1h, g2h, c, dinv16, wx, wh, b, wc):
    nb = _NP // _BN
    return pl.pallas_call(
        _cell_body,
        grid=(nb,),
        in_specs=[
            pl.BlockSpec((1, _BN, 8), lambda i: (t, i, 0)),
            pl.BlockSpec((1, _BN, 8), lambda i: (t, i, 0)),
            pl.BlockSpec((1, _BN, 8), lambda i: (t, i, 0)),
            pl.BlockSpec((_BN, _HID), lambda i: (i, 0)),
            pl.BlockSpec((2, _BN, 16), lambda i: (0, i, 0)),
            pl.BlockSpec((2, _BN, 16), lambda i: (0, i, 0)),
            pl.BlockSpec((_BN, _HID), lambda i: (i, 0)),
            pl.BlockSpec((_BN, 16), lambda i: (i, 0)),
            pl.BlockSpec((24, 128), lambda i: (0, 0)),
            pl.BlockSpec((96, 128), lambda i: (0, 0)),
            pl.BlockSpec((1, 128), lambda i: (0, 0)),
            pl.BlockSpec((3, _HID), lambda i: (0, 0)),
        ],
        out_specs=[
            pl.BlockSpec((_BN, _HID), lambda i: (i, 0)),
            pl.BlockSpec((_BN, _HID), lambda i: (i, 0)),
            pl.BlockSpec((2, _BN, 16), lambda i: (0, i, 0)),
        ],
        out_shape=[
            jax.ShapeDtypeStruct((_NP, _HID), jnp.float32),
            jax.ShapeDtypeStruct((_NP, _HID), jnp.float32),
            jax.ShapeDtypeStruct((2, _NP, 16), jnp.float32),
        ],
    )(x48, o1x, o2x, h, g1h, g2h, c, dinv16, wx, wh, b, wc)


def _cat_weights(p, prefix, cin):
    cols = []
    for g in ("i", "f", "c", "o"):
        w = p["W" + prefix + "_" + g]
        cols.append(jnp.concatenate([w[0] - w[2], -w[1], -2.0 * w[2]], axis=0))
    return jnp.concatenate(cols, axis=1)


def kernel(x, edge_index, h, c, params):
    p = params
    src = edge_index[0].astype(jnp.int32)
    dst = edge_index[1].astype(jnp.int32)
    self_m = src == dst
    n_i32 = jnp.int32(_N)
    srcg = jnp.where(self_m, n_i32, src)
    dstg = jnp.where(self_m, n_i32, dst)

    npad_e = _EPAD - _E
    padg = jnp.full((npad_e,), n_i32, jnp.int32)
    pads = (n_i32 + 1 + (jnp.arange(npad_e, dtype=jnp.int32)
                         % jnp.int32(_NP - _N - 1)))
    def lay(g, s):
        g = jnp.concatenate([g, padg]).reshape(_NTILES, _NB, 1, _EB)
        s = jnp.concatenate([s, pads]).reshape(_NTILES, _NB, 1, _EB)
        return jnp.concatenate([g, s], axis=2)
    idx_p = lay(srcg, dst)
    idx_d = lay(dstg, src)
    zz = jnp.zeros((_NP, 16), jnp.float32)

    h_pad = jnp.pad(h, ((0, _NP - _N), (0, 0)))
    c_pad = jnp.pad(c, ((0, _NP - _N), (0, 0)))
    x48 = x.reshape(_N, _C_IN, _T_IN).transpose(0, 2, 1).reshape(_N, 48)
    x48_pad = jnp.pad(x48, ((0, _NP - _N), (0, 0)))

    ones_tab = jnp.pad(jnp.ones((2, _N, 16), jnp.float32),
                       ((0, 0), (0, _NP - _N), (0, 0)))

    deg_out = _sc_prop(idx_d, zz, ones_tab)
    dinv16, mdinv2, uh, ux1, ux2 = _tc_prep(deg_out[0], h_pad, x48_pad)

    g1xa, g2xa = _sc_prop2(idx_p, zz, ux1, mdinv2)
    g1xb, g2xb = _sc_prop2(idx_p, zz, ux2, mdinv2)

    def to6(ga, gb):
        flat = jnp.concatenate([ga[0], ga[1], gb[0]], axis=1)
        return flat.reshape(_NP, _T_IN, _C_IN).transpose(1, 0, 2)
    o1x = to6(g1xa, g1xb)
    o2x = to6(g2xa, g2xb)
    x6 = x48_pad.reshape(_NP, _T_IN, _C_IN).transpose(1, 0, 2)

    wx = _cat_weights(p, "x", _C_IN)
    wh = _cat_weights(p, "h", _HID)
    b = jnp.concatenate([p["bx_" + g] + p["bh_" + g] + p["b_" + g]
                         for g in ("i", "f", "c", "o")]).reshape(1, 128)
    wc = jnp.stack([p["w_c_i"], p["w_c_f"], p["w_c_o"]])

    h_cur, c_cur, u_cur = h_pad, c_pad, uh
    hs, cs = [], []
    for t in range(_T_IN):
        for _ in range(_NLAYERS):
            g1h, g2h = _sc_prop2(idx_p, zz, u_cur, mdinv2)
            h_cur, c_cur, u_cur = _tc_cell(
                t, x6, o1x, o2x, h_cur, g1h, g2h, c_cur,
                dinv16, wx, wh, b, wc)
        hs.append(h_cur[:_N])
        cs.append(c_cur[:_N])
    return jnp.stack(hs), jnp.stack(cs)
